# baseline (speedup 1.0000x reference)
import functools
import math

import jax
import jax.numpy as jnp
from jax import lax
from jax.experimental import pallas as pl
from jax.experimental.pallas import tpu as pltpu
from jax.experimental.pallas import tpu_sc as plsc

NUM_CORES = 2
NUM_SUBCORES = 16
NUM_WORKERS = NUM_CORES * NUM_SUBCORES
LANES = 16

VOCAB = 1000000
EMB = 64
SCALE = math.sqrt(EMB)
SEQ = 200
BATCH = 4096

PANEL = 256
N_PANELS = VOCAB // PANEL
TAIL = VOCAB - N_PANELS * PANEL

CHUNK = 256
B_CHUNKS = BATCH // CHUNK
PER_TILE = (SEQ * B_CHUNKS) // NUM_WORKERS


def _tr_kernel(tabt_hbm, tail_hbm, pairs_hbm, in_v, out_v, sem_i, sem_o):
    wid = lax.axis_index("s") * NUM_CORES + lax.axis_index("c")
    lane_iota = lax.iota(jnp.int32, LANES)
    rows16 = [16 * q + lane_iota for q in range(4)]

    def do_panel(c0, p_rows):
        c0 = pl.multiple_of(c0, PANEL)
        pltpu.async_copy(tabt_hbm.at[:, pl.ds(c0, PANEL)], in_v, sem_i).wait()

        def p_body(p, _):
            for c0h in range(8):
                col = lane_iota * 0 + (2 * p + c0h // 4)
                vals = plsc.load_gather(in_v, [rows16[c0h % 4], col])
                out_v[p, pl.ds(16 * c0h, LANES)] = vals
            return ()

        lax.fori_loop(0, p_rows, p_body, (), unroll=False)
        pltpu.async_copy(out_v.at[pl.ds(0, p_rows)],
                         pairs_hbm.at[pl.ds(pl.multiple_of(c0 // 2, PANEL // 2), p_rows)],
                         sem_o).wait()

    n_mine = (N_PANELS + NUM_WORKERS - 1 - wid) // NUM_WORKERS

    def body(j, _):
        k = wid + NUM_WORKERS * j
        do_panel(k * PANEL, PANEL // 2)
        return ()

    lax.fori_loop(0, n_mine, body, (), unroll=False)

    @pl.when(wid == 0)
    def _():
        pltpu.async_copy(tail_hbm, out_v.at[pl.ds(0, TAIL // 2)], sem_i).wait()
        pltpu.async_copy(out_v.at[pl.ds(0, TAIL // 2)],
                         pairs_hbm.at[pl.ds(N_PANELS * PANEL // 2, TAIL // 2)],
                         sem_o).wait()


def _emb_kernel(pairs_hbm, toks_hbm, out_hbm, tok_v, idx_v, par_v, rows_v,
                outt_v, sem_i, sem_g, sem_o):
    wid = lax.axis_index("s") * NUM_CORES + lax.axis_index("c")
    lane_iota = lax.iota(jnp.int32, LANES)

    def body(i, _):
        k = wid * PER_TILE + i
        s = k // B_CHUNKS
        b0 = pl.multiple_of((k % B_CHUNKS) * CHUNK, CHUNK)

        pltpu.async_copy(toks_hbm.at[s, pl.ds(b0, CHUNK)], tok_v, sem_i).wait()
        for j in range(CHUNK // LANES):
            t = tok_v[pl.ds(j * LANES, LANES)]
            idx_v[pl.ds(j * LANES, LANES)] = t >> 1
            par_v[pl.ds(j * LANES, LANES)] = (t & 1) * EMB

        pltpu.async_copy(pairs_hbm.at[idx_v], rows_v, sem_g).wait()

        def rg_body(rg, _):
            row_ids = rg * LANES + lane_iota
            col0 = par_v[pl.ds(rg * LANES, LANES)]
            for f in range(EMB):
                vals = plsc.load_gather(rows_v, [row_ids, col0 + f])
                outt_v[f, pl.ds(rg * LANES, LANES)] = vals * SCALE
            return ()

        lax.fori_loop(0, CHUNK // LANES, rg_body, (), unroll=False)

        pltpu.async_copy(outt_v, out_hbm.at[s, :, pl.ds(b0, CHUNK)],
                         sem_o).wait()
        return ()

    lax.fori_loop(0, PER_TILE, body, (), unroll=False)


@jax.jit
def kernel(tokens, embedding):
    tabt = jnp.transpose(embedding)
    tail2 = jnp.reshape(embedding[N_PANELS * PANEL:], (TAIL // 2, 128))
    toks2 = jnp.transpose(tokens).astype(jnp.int32)

    mesh = plsc.VectorSubcoreMesh(core_axis_name="c", subcore_axis_name="s")
    params = pltpu.CompilerParams(needs_layout_passes=False)

    pairs = pl.kernel(
        _tr_kernel,
        out_type=jax.ShapeDtypeStruct((VOCAB // 2, 128), jnp.float32),
        mesh=mesh,
        compiler_params=params,
        scratch_types=[
            pltpu.VMEM((EMB, PANEL), jnp.float32),
            pltpu.VMEM((PANEL // 2, 128), jnp.float32),
            pltpu.SemaphoreType.DMA,
            pltpu.SemaphoreType.DMA,
        ],
    )(tabt, tail2)

    out = pl.kernel(
        _emb_kernel,
        out_type=jax.ShapeDtypeStruct((SEQ, EMB, BATCH), jnp.float32),
        mesh=mesh,
        compiler_params=params,
        scratch_types=[
            pltpu.VMEM((CHUNK,), jnp.int32),
            pltpu.VMEM((CHUNK,), jnp.int32),
            pltpu.VMEM((CHUNK,), jnp.int32),
            pltpu.VMEM((CHUNK, 128), jnp.float32),
            pltpu.VMEM((EMB, CHUNK), jnp.float32),
            pltpu.SemaphoreType.DMA,
            pltpu.SemaphoreType.DMA,
            pltpu.SemaphoreType.DMA,
        ],
    )(pairs, toks2)
    return jnp.transpose(out, (2, 0, 1))


# speedup vs baseline: 1.1903x; 1.1903x over previous
import functools
import math

import jax
import jax.numpy as jnp
from jax import lax
from jax.experimental import pallas as pl
from jax.experimental.pallas import tpu as pltpu
from jax.experimental.pallas import tpu_sc as plsc

NUM_CORES = 2
NUM_SUBCORES = 16
NUM_WORKERS = NUM_CORES * NUM_SUBCORES
LANES = 16

VOCAB = 1000000
EMB = 64
SCALE = math.sqrt(EMB)
SEQ = 200
BATCH = 4096

PANEL = 256
N_PANELS = VOCAB // PANEL
TAIL = VOCAB - N_PANELS * PANEL
TR_W = 130

CHUNK = 256
OUT_W = CHUNK + 1
B_CHUNKS = BATCH // CHUNK
PER_TILE = (SEQ * B_CHUNKS) // NUM_WORKERS


def _tr_kernel(tabt_hbm, tail_hbm, pairs_hbm, in_v, out_v, sem_i, sem_o):
    wid = lax.axis_index("s") * NUM_CORES + lax.axis_index("c")
    lane_iota = lax.iota(jnp.int32, LANES)
    rowp = lane_iota >> 1
    colp = (lane_iota & 1) * EMB
    n_mine = (N_PANELS + NUM_WORKERS - 1 - wid) // NUM_WORKERS

    def start_panel(j, slot):
        k = wid + NUM_WORKERS * j
        c0 = pl.multiple_of(k * PANEL, PANEL)
        return pltpu.async_copy(tabt_hbm.at[:, pl.ds(c0, PANEL)],
                                in_v.at[slot], sem_i)

    def compute_and_flush(j, slot):
        k = wid + NUM_WORKERS * j

        def m_body(m, _):
            rows = 8 * m + rowp
            for f in range(EMB):
                vals = in_v[slot, f, pl.ds(16 * m, LANES)]
                plsc.store_scatter(out_v, [rows, colp + f], vals)
            return ()

        lax.fori_loop(0, PANEL // LANES, m_body, (), unroll=False)
        pltpu.async_copy(
            out_v.at[:, pl.ds(0, 128)],
            pairs_hbm.at[pl.ds(pl.multiple_of(k * (PANEL // 2), PANEL // 2),
                               PANEL // 2)],
            sem_o).wait()

    @pl.when(n_mine > 0)
    def _():
        start_panel(0, 0)

        def body(j, _):
            slot = lax.rem(j, 2)

            @pl.when(j + 1 < n_mine)
            def _():
                start_panel(j + 1, 1 - slot)

            pltpu.make_async_copy(tabt_hbm.at[:, pl.ds(0, PANEL)],
                                  in_v.at[slot], sem_i).wait()
            compute_and_flush(j, slot)
            return ()

        lax.fori_loop(0, n_mine, body, (), unroll=False)

    @pl.when(wid == 0)
    def _():
        pltpu.async_copy(tail_hbm, out_v.at[pl.ds(0, TAIL // 2),
                                            pl.ds(0, 128)], sem_i).wait()
        pltpu.async_copy(out_v.at[pl.ds(0, TAIL // 2), pl.ds(0, 128)],
                         pairs_hbm.at[pl.ds(N_PANELS * PANEL // 2, TAIL // 2)],
                         sem_o).wait()


def _emb_kernel(pairs_hbm, toks_hbm, out_hbm, tok0_v, tok1_v, idx0_v,
                idx1_v, par0_v, par1_v, rows0_v, rows1_v, outt_v,
                sem_i, sem_g0, sem_g1, sem_o):
    wid = lax.axis_index("s") * NUM_CORES + lax.axis_index("c")
    lane_iota = lax.iota(jnp.int32, LANES)
    fcols = [16 * j + lane_iota for j in range(EMB // LANES)]
    bufs = [(tok0_v, idx0_v, par0_v, rows0_v, sem_g0),
            (tok1_v, idx1_v, par1_v, rows1_v, sem_g1)]

    def stage_and_gather(i, b):
        tok_v, idx_v, par_v, rows_v, sem_g = bufs[b]
        k = wid * PER_TILE + i
        s = k // B_CHUNKS
        b0 = pl.multiple_of((k % B_CHUNKS) * CHUNK, CHUNK)
        pltpu.async_copy(toks_hbm.at[s, pl.ds(b0, CHUNK)], tok_v, sem_i).wait()
        for j in range(CHUNK // LANES):
            t = tok_v[pl.ds(j * LANES, LANES)]
            idx_v[pl.ds(j * LANES, LANES)] = t >> 1
            par_v[pl.ds(j * LANES, LANES)] = (t & 1) * EMB
        pltpu.async_copy(pairs_hbm.at[idx_v], rows_v, sem_g)

    def compute_and_flush(i, b):
        tok_v, idx_v, par_v, rows_v, sem_g = bufs[b]
        k = wid * PER_TILE + i
        s = k // B_CHUNKS
        b0 = pl.multiple_of((k % B_CHUNKS) * CHUNK, CHUNK)
        pltpu.make_async_copy(pairs_hbm.at[idx_v], rows_v, sem_g).wait()

        def r_body(r, _):
            p64 = par_v[pl.ds(r, LANES)][0]
            col = lane_iota * 0 + r
            for j in range(EMB // LANES):
                vals = rows_v[r, pl.ds(p64 + 16 * j, LANES)] * SCALE
                plsc.store_scatter(outt_v, [fcols[j], col], vals)
            return ()

        lax.fori_loop(0, CHUNK, r_body, (), unroll=False)
        pltpu.async_copy(outt_v.at[:, pl.ds(0, CHUNK)],
                         out_hbm.at[s, :, pl.ds(b0, CHUNK)], sem_o).wait()

    stage_and_gather(0, 0)
    stage_and_gather(1, 1)

    def body(it, _):
        for b in range(2):
            i = 2 * it + b
            compute_and_flush(i, b)

            @pl.when(i + 2 < PER_TILE)
            def _():
                stage_and_gather(i + 2, b)
        return ()

    lax.fori_loop(0, PER_TILE // 2, body, (), unroll=False)


@jax.jit
def kernel(tokens, embedding):
    tabt = jnp.transpose(embedding)
    tail2 = jnp.reshape(embedding[N_PANELS * PANEL:], (TAIL // 2, 128))
    toks2 = jnp.transpose(tokens).astype(jnp.int32)

    mesh = plsc.VectorSubcoreMesh(core_axis_name="c", subcore_axis_name="s")
    params = pltpu.CompilerParams(needs_layout_passes=False)

    pairs = pl.kernel(
        _tr_kernel,
        out_type=jax.ShapeDtypeStruct((VOCAB // 2, 128), jnp.float32),
        mesh=mesh,
        compiler_params=params,
        scratch_types=[
            pltpu.VMEM((2, EMB, PANEL), jnp.float32),
            pltpu.VMEM((PANEL // 2, TR_W), jnp.float32),
            pltpu.SemaphoreType.DMA,
            pltpu.SemaphoreType.DMA,
        ],
    )(tabt, tail2)

    out = pl.kernel(
        _emb_kernel,
        out_type=jax.ShapeDtypeStruct((SEQ, EMB, BATCH), jnp.float32),
        mesh=mesh,
        compiler_params=params,
        scratch_types=[
            pltpu.VMEM((CHUNK,), jnp.int32),
            pltpu.VMEM((CHUNK,), jnp.int32),
            pltpu.VMEM((CHUNK,), jnp.int32),
            pltpu.VMEM((CHUNK,), jnp.int32),
            pltpu.VMEM((CHUNK + LANES,), jnp.int32),
            pltpu.VMEM((CHUNK + LANES,), jnp.int32),
            pltpu.VMEM((CHUNK, 128), jnp.float32),
            pltpu.VMEM((CHUNK, 128), jnp.float32),
            pltpu.VMEM((EMB, OUT_W), jnp.float32),
            pltpu.SemaphoreType.DMA,
            pltpu.SemaphoreType.DMA,
            pltpu.SemaphoreType.DMA,
            pltpu.SemaphoreType.DMA,
        ],
    )(pairs, toks2)
    return jnp.transpose(out, (2, 0, 1))


# speedup vs baseline: 1.9318x; 1.6230x over previous
import functools
import math

import jax
import jax.numpy as jnp
from jax import lax
from jax.experimental import pallas as pl
from jax.experimental.pallas import tpu as pltpu
from jax.experimental.pallas import tpu_sc as plsc

NUM_CORES = 2
NUM_SUBCORES = 16
NUM_WORKERS = NUM_CORES * NUM_SUBCORES
LANES = 16

VOCAB = 1000000
EMB = 64
PADW = 128
SCALE = math.sqrt(EMB)
SEQ = 200
BATCH = 4096

CHUNK = 256
OUT_W = CHUNK + 1
B_CHUNKS = BATCH // CHUNK
PER_TILE = (SEQ * B_CHUNKS) // NUM_WORKERS


def _emb_kernel(table_hbm, toks_hbm, out_hbm, idx0_v, idx1_v, rows0_v,
                rows1_v, outt_v, sem_i, sem_g0, sem_g1, sem_o):
    wid = lax.axis_index("s") * NUM_CORES + lax.axis_index("c")
    lane_iota = lax.iota(jnp.int32, LANES)
    fcols = [16 * j + lane_iota for j in range(EMB // LANES)]
    bufs = [(idx0_v, rows0_v, sem_g0), (idx1_v, rows1_v, sem_g1)]

    def chunk_coords(i):
        k = wid * PER_TILE + i
        s = k // B_CHUNKS
        b0 = pl.multiple_of((k % B_CHUNKS) * CHUNK, CHUNK)
        return s, b0

    def stage_and_gather(i, b):
        idx_v, rows_v, sem_g = bufs[b]
        s, b0 = chunk_coords(i)
        pltpu.async_copy(toks_hbm.at[s, pl.ds(b0, CHUNK)], idx_v, sem_i).wait()
        pltpu.async_copy(table_hbm.at[idx_v], rows_v, sem_g)

    def compute_and_flush(i, b):
        idx_v, rows_v, sem_g = bufs[b]
        s, b0 = chunk_coords(i)
        pltpu.make_async_copy(table_hbm.at[idx_v], rows_v, sem_g).wait()

        def r_body(r, _):
            col = lane_iota * 0 + r
            for j in range(EMB // LANES):
                vals = rows_v[r, pl.ds(16 * j, LANES)] * SCALE
                plsc.store_scatter(outt_v, [fcols[j], col], vals)
            return ()

        lax.fori_loop(0, CHUNK, r_body, (), unroll=4)
        pltpu.async_copy(outt_v.at[:, pl.ds(0, CHUNK)],
                         out_hbm.at[s, :, pl.ds(b0, CHUNK)], sem_o).wait()

    stage_and_gather(0, 0)
    stage_and_gather(1, 1)

    def body(it, _):
        for b in range(2):
            i = 2 * it + b
            compute_and_flush(i, b)

            @pl.when(i + 2 < PER_TILE)
            def _():
                stage_and_gather(i + 2, b)
        return ()

    lax.fori_loop(0, PER_TILE // 2, body, (), unroll=False)


@jax.jit
def kernel(tokens, embedding):
    table_p = jnp.pad(embedding, ((0, 0), (0, PADW - EMB)))
    toks2 = jnp.transpose(tokens).astype(jnp.int32)

    mesh = plsc.VectorSubcoreMesh(core_axis_name="c", subcore_axis_name="s")
    out = pl.kernel(
        _emb_kernel,
        out_type=jax.ShapeDtypeStruct((SEQ, EMB, BATCH), jnp.float32),
        mesh=mesh,
        compiler_params=pltpu.CompilerParams(needs_layout_passes=False),
        scratch_types=[
            pltpu.VMEM((CHUNK,), jnp.int32),
            pltpu.VMEM((CHUNK,), jnp.int32),
            pltpu.VMEM((CHUNK, PADW), jnp.float32),
            pltpu.VMEM((CHUNK, PADW), jnp.float32),
            pltpu.VMEM((EMB, OUT_W), jnp.float32),
            pltpu.SemaphoreType.DMA,
            pltpu.SemaphoreType.DMA,
            pltpu.SemaphoreType.DMA,
            pltpu.SemaphoreType.DMA,
        ],
    )(table_p, toks2)
    return jnp.transpose(out, (2, 0, 1))


# speedup vs baseline: 2.5453x; 1.3176x over previous
import functools
import math

import jax
import jax.numpy as jnp
from jax import lax
from jax.experimental import pallas as pl
from jax.experimental.pallas import tpu as pltpu
from jax.experimental.pallas import tpu_sc as plsc

NUM_CORES = 2
NUM_SUBCORES = 16
NUM_WORKERS = NUM_CORES * NUM_SUBCORES
LANES = 16

EMB = 64
SCALE = math.sqrt(EMB)
CHUNK = 512


def _emb_kernel(table_hbm, idx_hbm, out_hbm, idx_v, rows_v, sem):
    wid = lax.axis_index("s") * NUM_CORES + lax.axis_index("c")
    b_per_w = idx_hbm.shape[0] // NUM_WORKERS
    n_chunks = b_per_w // CHUNK
    base = wid * b_per_w

    def body(g, _):
        off = base + g * CHUNK
        pltpu.sync_copy(idx_hbm.at[pl.ds(off, CHUNK)], idx_v)
        pltpu.async_copy(table_hbm.at[idx_v], rows_v, sem).wait()

        def scale_row(r, _):
            for j in range(EMB // LANES):
                sl = pl.ds(j * LANES, LANES)
                rows_v[r, sl] = rows_v[r, sl] * SCALE
            return ()

        lax.fori_loop(0, CHUNK, scale_row, (), unroll=2)
        pltpu.sync_copy(rows_v, out_hbm.at[pl.ds(off, CHUNK)])
        return ()

    lax.fori_loop(0, n_chunks, body, (), unroll=False)


@jax.jit
def kernel(tokens, embedding):
    B = tokens.shape[0] * tokens.shape[1]
    idx = tokens.reshape((B,)).astype(jnp.int32)

    mesh = plsc.VectorSubcoreMesh(core_axis_name="c", subcore_axis_name="s")
    out = pl.kernel(
        _emb_kernel,
        out_type=jax.ShapeDtypeStruct((B, EMB), jnp.float32),
        mesh=mesh,
        compiler_params=pltpu.CompilerParams(use_tc_tiling_on_sc=False),
        scratch_types=[
            pltpu.VMEM((CHUNK,), jnp.int32),
            pltpu.VMEM((CHUNK, EMB), jnp.float32),
            pltpu.SemaphoreType.DMA,
        ],
    )(embedding, idx)
    return out.reshape(tokens.shape + (EMB,))


# speedup vs baseline: 2.5536x; 1.0033x over previous
import functools
import math

import jax
import jax.numpy as jnp
from jax import lax
from jax.experimental import pallas as pl
from jax.experimental.pallas import tpu as pltpu
from jax.experimental.pallas import tpu_sc as plsc

NUM_CORES = 2
NUM_SUBCORES = 16
NUM_WORKERS = NUM_CORES * NUM_SUBCORES
LANES = 16

EMB = 64
SCALE = math.sqrt(EMB)
CHUNK = 512


def _emb_kernel(table_hbm, idx_hbm, out_hbm, idx_v, rows_v, sem):
    wid = lax.axis_index("s") * NUM_CORES + lax.axis_index("c")
    b_per_w = idx_hbm.shape[0] // NUM_WORKERS
    n_chunks = b_per_w // CHUNK
    base = wid * b_per_w

    def body(g, _):
        off = base + g * CHUNK
        pltpu.sync_copy(idx_hbm.at[pl.ds(off, CHUNK)], idx_v)
        pltpu.async_copy(table_hbm.at[idx_v], rows_v, sem).wait()

        def scale_row(r, _):
            for j in range(EMB // LANES):
                sl = pl.ds(j * LANES, LANES)
                rows_v[r, sl] = rows_v[r, sl] * SCALE
            return ()

        lax.fori_loop(0, CHUNK, scale_row, (), unroll=8)
        pltpu.sync_copy(rows_v, out_hbm.at[pl.ds(off, CHUNK)])
        return ()

    lax.fori_loop(0, n_chunks, body, (), unroll=False)


@jax.jit
def kernel(tokens, embedding):
    B = tokens.shape[0] * tokens.shape[1]
    idx = tokens.reshape((B,)).astype(jnp.int32)

    mesh = plsc.VectorSubcoreMesh(core_axis_name="c", subcore_axis_name="s")
    out = pl.kernel(
        _emb_kernel,
        out_type=jax.ShapeDtypeStruct((B, EMB), jnp.float32),
        mesh=mesh,
        compiler_params=pltpu.CompilerParams(use_tc_tiling_on_sc=False),
        scratch_types=[
            pltpu.VMEM((CHUNK,), jnp.int32),
            pltpu.VMEM((CHUNK, EMB), jnp.float32),
            pltpu.SemaphoreType.DMA,
        ],
    )(embedding, idx)
    return out.reshape(tokens.shape + (EMB,))
